# baseline (speedup 1.0000x reference)
import functools

import jax
import jax.numpy as jnp
from jax import lax
from jax.experimental import pallas as pl
from jax.experimental.pallas import tpu as pltpu
from jax.experimental.pallas import tpu_sc as plsc

B0, B1 = 16384, 50
NUM_TOKENS = B0 * B1
EMBED = 32
NROWS = 1000000
PACK = 128 // EMBED
NPACKED = NROWS // PACK

NC = 2
NS = 16
NW = NC * NS

DP_CHUNK_P = 1000

ROWS_PER_W = B0 // NW
RCHUNK = 32
N_CHUNKS = ROWS_PER_W // RCHUNK
CHUNK = RCHUNK * B1
L = 16

_MESH = plsc.VectorSubcoreMesh(core_axis_name="c", subcore_axis_name="s")


def _tc_depad_body(a_ref, out_ref):
    k = pl.program_id(1)
    for kk in range(PACK):
        @pl.when(k == kk)
        def _():
            out_ref[:, 32 * kk:32 * kk + 32] = a_ref[...]


def _make_depad_kernel():
    nblk = NPACKED // DP_CHUNK_P

    return pl.pallas_call(
        _tc_depad_body,
        grid=(nblk, PACK),
        in_specs=[
            pl.BlockSpec((DP_CHUNK_P, EMBED), lambda i, k: (i + k * nblk, 0)),
        ],
        out_specs=pl.BlockSpec((DP_CHUNK_P, 128), lambda i, k: (i, 0)),
        out_shape=jax.ShapeDtypeStruct((NPACKED, 128), jnp.float32),
    )


def _make_emb_kernel(b0):
    rows_per_w = b0 // NW
    n_chunks = rows_per_w // RCHUNK

    @functools.partial(
        pl.kernel,
        mesh=_MESH,
        out_type=jax.ShapeDtypeStruct((b0, B1, EMBED), jnp.float32),
        scratch_types=[
            pltpu.VMEM((CHUNK,), jnp.int32),
            pltpu.VMEM((CHUNK, EMBED), jnp.float32),
            pltpu.SemaphoreType.DMA,
            pltpu.SemaphoreType.DMA,
        ],
        compiler_params=pltpu.CompilerParams(use_tc_tiling_on_sc=False),
    )
    def emb_kernel(idx_hbm, table_hbm, out_hbm, idx_v, rows_v, gsem, osem):
        wid = lax.axis_index("s") * NC + lax.axis_index("c")
        row_base = wid * rows_per_w

        def body(i, _):
            row_off = row_base + i * RCHUNK
            off = row_off * B1
            pltpu.sync_copy(idx_hbm.at[pl.ds(off, CHUNK)], idx_v)
            pltpu.async_copy(table_hbm.at[idx_v], rows_v, gsem).wait()
            handles = [
                pltpu.async_copy(
                    rows_v.at[pl.ds(j * B1, B1)], out_hbm.at[row_off + j], osem
                )
                for j in range(RCHUNK)
            ]
            for h in handles:
                h.wait()
            return 0

        lax.fori_loop(0, n_chunks, body, 0)

    return emb_kernel


NSPLIT = 2
B0H = B0 // NSPLIT
_emb = _make_emb_kernel(B0H)


@jax.jit
def kernel(token, table):
    flat = token.reshape(-1)
    q = flat // NPACKED
    flat = (flat - q * NPACKED) * PACK + q
    nh = NPACKED // 2
    packed_halves = [
        jnp.concatenate(
            [table[k * NPACKED + h * nh: k * NPACKED + (h + 1) * nh]
             for k in range(PACK)], axis=1
        )
        for h in range(2)
    ]
    packed = jnp.concatenate(packed_halves, axis=0)
    tab = packed.reshape(NROWS, EMBED)
    n = B0H * B1
    halves = [
        _emb(lax.dynamic_slice_in_dim(flat, h * n, n), tab)
        for h in range(NSPLIT)
    ]
    return jnp.concatenate(halves, axis=0)


# speedup vs baseline: 1.0788x; 1.0788x over previous
import functools

import jax
import jax.numpy as jnp
from jax import lax
from jax.experimental import pallas as pl
from jax.experimental.pallas import tpu as pltpu
from jax.experimental.pallas import tpu_sc as plsc

B0, B1 = 16384, 50
NUM_TOKENS = B0 * B1
EMBED = 32
NROWS = 1000000
PACK = 128 // EMBED
NPACKED = NROWS // PACK

NC = 2
NS = 16
NW = NC * NS

DP_CHUNK_P = 1000

ROWS_PER_W = B0 // NW
RCHUNK = 32
N_CHUNKS = ROWS_PER_W // RCHUNK
CHUNK = RCHUNK * B1
L = 16

_MESH = plsc.VectorSubcoreMesh(core_axis_name="c", subcore_axis_name="s")


def _tc_depad_body(a_ref, out_ref):
    k = pl.program_id(1)
    for kk in range(PACK):
        @pl.when(k == kk)
        def _():
            out_ref[:, 32 * kk:32 * kk + 32] = a_ref[...]


def _make_depad_kernel():
    nblk = NPACKED // DP_CHUNK_P

    return pl.pallas_call(
        _tc_depad_body,
        grid=(nblk, PACK),
        in_specs=[
            pl.BlockSpec((DP_CHUNK_P, EMBED), lambda i, k: (i + k * nblk, 0)),
        ],
        out_specs=pl.BlockSpec((DP_CHUNK_P, 128), lambda i, k: (i, 0)),
        out_shape=jax.ShapeDtypeStruct((NPACKED, 128), jnp.float32),
    )


def _make_emb_kernel(b0):
    rows_per_w = b0 // NW
    n_chunks = rows_per_w // RCHUNK

    @functools.partial(
        pl.kernel,
        mesh=_MESH,
        out_type=jax.ShapeDtypeStruct((b0, B1, EMBED), jnp.float32),
        scratch_types=[
            pltpu.VMEM((CHUNK,), jnp.int32),
            pltpu.VMEM((CHUNK, EMBED), jnp.float32),
            pltpu.SemaphoreType.DMA,
            pltpu.SemaphoreType.DMA,
        ],
        compiler_params=pltpu.CompilerParams(use_tc_tiling_on_sc=False),
    )
    def emb_kernel(idx_hbm, table_hbm, out_hbm, idx_v, rows_v, gsem, osem):
        wid = lax.axis_index("s") * NC + lax.axis_index("c")
        row_base = wid * rows_per_w

        def body(i, _):
            row_off = row_base + i * RCHUNK
            off = row_off * B1
            pltpu.sync_copy(idx_hbm.at[pl.ds(off, CHUNK)], idx_v)
            pltpu.async_copy(table_hbm.at[idx_v], rows_v, gsem).wait()
            handles = [
                pltpu.async_copy(
                    rows_v.at[pl.ds(j * B1, B1)], out_hbm.at[row_off + j], osem
                )
                for j in range(RCHUNK)
            ]
            for h in handles:
                h.wait()
            return 0

        lax.fori_loop(0, n_chunks, body, 0)

    return emb_kernel


NSPLIT = 4
B0H = B0 // NSPLIT
_emb = _make_emb_kernel(B0H)


@jax.jit
def kernel(token, table):
    flat = token.reshape(-1)
    q = flat // NPACKED
    flat = (flat - q * NPACKED) * PACK + q
    packed = jnp.concatenate(
        [table[k * NPACKED:(k + 1) * NPACKED] for k in range(PACK)], axis=1
    )
    tab = packed.reshape(NROWS, EMBED)
    n = B0H * B1
    halves = [
        _emb(lax.dynamic_slice_in_dim(flat, h * n, n), tab)
        for h in range(NSPLIT)
    ]
    return jnp.concatenate(halves, axis=0)


# speedup vs baseline: 1.0825x; 1.0034x over previous
import functools

import jax
import jax.numpy as jnp
from jax import lax
from jax.experimental import pallas as pl
from jax.experimental.pallas import tpu as pltpu
from jax.experimental.pallas import tpu_sc as plsc

B0, B1 = 16384, 50
NUM_TOKENS = B0 * B1
EMBED = 32
NROWS = 1000000
PACK = 128 // EMBED
NPACKED = NROWS // PACK

NC = 2
NS = 16
NW = NC * NS

DP_CHUNK_P = 1000

ROWS_PER_W = B0 // NW
RCHUNK = 32
N_CHUNKS = ROWS_PER_W // RCHUNK
CHUNK = RCHUNK * B1
L = 16

_MESH = plsc.VectorSubcoreMesh(core_axis_name="c", subcore_axis_name="s")


def _tc_depad_body(a_ref, out_ref):
    k = pl.program_id(1)
    for kk in range(PACK):
        @pl.when(k == kk)
        def _():
            out_ref[:, 32 * kk:32 * kk + 32] = a_ref[...]


def _make_depad_kernel():
    nblk = NPACKED // DP_CHUNK_P

    return pl.pallas_call(
        _tc_depad_body,
        grid=(nblk, PACK),
        in_specs=[
            pl.BlockSpec((DP_CHUNK_P, EMBED), lambda i, k: (i + k * nblk, 0)),
        ],
        out_specs=pl.BlockSpec((DP_CHUNK_P, 128), lambda i, k: (i, 0)),
        out_shape=jax.ShapeDtypeStruct((NPACKED, 128), jnp.float32),
    )


def _make_emb_kernel(b0):
    rows_per_w = b0 // NW
    n_chunks = rows_per_w // RCHUNK

    @functools.partial(
        pl.kernel,
        mesh=_MESH,
        out_type=jax.ShapeDtypeStruct((b0, B1, EMBED), jnp.float32),
        scratch_types=[
            pltpu.VMEM((CHUNK,), jnp.int32),
            pltpu.VMEM((CHUNK, EMBED), jnp.float32),
            pltpu.SemaphoreType.DMA,
            pltpu.SemaphoreType.DMA,
        ],
        compiler_params=pltpu.CompilerParams(use_tc_tiling_on_sc=False),
    )
    def emb_kernel(idx_hbm, table_hbm, out_hbm, idx_v, rows_v, gsem, osem):
        wid = lax.axis_index("s") * NC + lax.axis_index("c")
        row_base = wid * rows_per_w

        def body(i, _):
            row_off = row_base + i * RCHUNK
            off = row_off * B1
            pltpu.sync_copy(idx_hbm.at[pl.ds(off, CHUNK)], idx_v)
            pltpu.async_copy(table_hbm.at[idx_v], rows_v, gsem).wait()
            handles = [
                pltpu.async_copy(
                    rows_v.at[pl.ds(j * B1, B1)], out_hbm.at[row_off + j], osem
                )
                for j in range(RCHUNK)
            ]
            for h in handles:
                h.wait()
            return 0

        lax.fori_loop(0, n_chunks, body, 0)

    return emb_kernel


NSPLIT = 8
B0H = B0 // NSPLIT
_emb = _make_emb_kernel(B0H)


@jax.jit
def kernel(token, table):
    flat = token.reshape(-1)
    q = flat // NPACKED
    flat = (flat - q * NPACKED) * PACK + q
    packed = jnp.concatenate(
        [table[k * NPACKED:(k + 1) * NPACKED] for k in range(PACK)], axis=1
    )
    tab = packed.reshape(NROWS, EMBED)
    n = B0H * B1
    halves = [
        _emb(lax.dynamic_slice_in_dim(flat, h * n, n), tab)
        for h in range(NSPLIT)
    ]
    return jnp.concatenate(halves, axis=0)
